# Initial kernel scaffold; baseline (speedup 1.0000x reference)
import jax
import jax.numpy as jnp
from jax.experimental import pallas as pl


def kernel(input_ids, mode_embeds, word_embeddings, position_embeddings, ln_weight, ln_bias):
    raise NotImplementedError("write your pallas kernel here")


# speedup vs baseline: 1.2430x; 1.2430x over previous
import functools

import jax
import jax.numpy as jnp
from jax import lax
from jax.experimental import pallas as pl
from jax.experimental.pallas import tpu as pltpu
from jax.experimental.pallas import tpu_sc as plsc

HIDDEN = 128
MAX_POS = 512
EPS = 1e-12

_info = plsc.get_sparse_core_info()
_NC, _NS, _L = _info.num_cores, _info.num_subcores, _info.num_lanes
_NW = _NC * _NS
_T = 64
_HC = HIDDEN // 16


_GATHER_DNUMS = lax.GatherDimensionNumbers(
    offset_dims=(), collapsed_slice_dims=(0,), start_index_map=(0,))


def _lane_permute(v, perm):
    return lax.gather(v, perm[:, None], _GATHER_DNUMS, slice_sizes=(1,),
                      mode=lax.GatherScatterMode.PROMISE_IN_BOUNDS)


def _allreduce_sum(v):
    idx = lax.iota(jnp.int32, 16)
    for d in (1, 2, 4, 8):
        v = v + _lane_permute(v, lax.bitwise_xor(idx, jnp.int32(d)))
    return v


def _rsqrt_nr(x):
    i = lax.bitcast_convert_type(x, jnp.int32)
    i = jnp.int32(0x5F3759DF) - lax.shift_right_logical(i, 1)
    y = lax.bitcast_convert_type(i, jnp.float32)
    half = x * jnp.float32(0.5)
    for _ in range(3):
        y = y * (jnp.float32(1.5) - half * y * y)
    return y


def _make_sc_call(N):
    JW = N // _NW // _T
    mesh = plsc.VectorSubcoreMesh(core_axis_name="c", subcore_axis_name="s")

    @functools.partial(
        pl.kernel,
        out_type=jax.ShapeDtypeStruct((N, HIDDEN), jnp.float32),
        mesh=mesh,
        scratch_types=[
            pltpu.VMEM((JW, _T), jnp.int32),
            pltpu.VMEM((MAX_POS, HIDDEN), jnp.float32),
            pltpu.VMEM((HIDDEN,), jnp.float32),
            pltpu.VMEM((HIDDEN,), jnp.float32),
            pltpu.VMEM((_T, HIDDEN), jnp.float32),
            pltpu.VMEM((_T, HIDDEN), jnp.float32),
            pltpu.VMEM((_T, HIDDEN), jnp.float32),
            pltpu.SemaphoreType.DMA,
        ],
    )
    def sc_call(ids_hbm, mode_hbm, table_hbm, pos_hbm, w_hbm, b_hbm, out_hbm,
                idx_v, pos_v, w_v, b_v, rows_v, acc_v, out_v, sem):
        wid = lax.axis_index("s") * _NC + lax.axis_index("c")
        base = wid * (N // _NW)

        pltpu.sync_copy(ids_hbm.at[wid], idx_v)
        pltpu.sync_copy(pos_hbm, pos_v)
        pltpu.sync_copy(w_hbm, w_v)
        pltpu.sync_copy(b_hbm, b_v)

        ws = [w_v[pl.ds(16 * c, 16)] for c in range(_HC)]
        bs = [b_v[pl.ds(16 * c, 16)] for c in range(_HC)]

        def jbody(j, carry):
            t0 = base + j * _T
            pltpu.async_copy(table_hbm.at[idx_v.at[j]], rows_v, sem).wait()
            pltpu.sync_copy(mode_hbm.at[pl.ds(t0, _T)], acc_v)
            s0 = lax.rem(j * _T, MAX_POS)

            def tbody(t, c2):
                xs = [acc_v[t, pl.ds(16 * c, 16)]
                      + rows_v[t, pl.ds(16 * c, 16)]
                      + pos_v[s0 + t, pl.ds(16 * c, 16)]
                      for c in range(_HC)]
                s = xs[0]
                for c in range(1, _HC):
                    s = s + xs[c]
                mean_v = _allreduce_sum(s) * jnp.float32(1.0 / HIDDEN)
                d = [x - mean_v for x in xs]
                sq = d[0] * d[0]
                for c in range(1, _HC):
                    sq = sq + d[c] * d[c]
                var_v = _allreduce_sum(sq) * jnp.float32(1.0 / HIDDEN)
                rstd = _rsqrt_nr(var_v + jnp.float32(EPS))
                for c in range(_HC):
                    out_v[t, pl.ds(16 * c, 16)] = d[c] * rstd * ws[c] + bs[c]
                return c2

            lax.fori_loop(0, _T, tbody, 0, unroll=2)
            pltpu.sync_copy(out_v, out_hbm.at[pl.ds(t0, _T)])
            return carry

        lax.fori_loop(0, JW, jbody, 0)

    return sc_call


def kernel(input_ids, mode_embeds, word_embeddings, position_embeddings,
           ln_weight, ln_bias):
    B, S = input_ids.shape
    N = B * S
    ids = input_ids.astype(jnp.int32).reshape(_NW, N // _NW // _T, _T)
    mode = mode_embeds.reshape(N, HIDDEN)
    out = _make_sc_call(N)(ids, mode, word_embeddings, position_embeddings,
                           ln_weight, ln_bias)
    return out.reshape(B, S, HIDDEN)
